# baseline (speedup 1.0000x reference)
import functools

import jax
import jax.numpy as jnp
from jax import lax
from jax.experimental import pallas as pl
from jax.experimental.pallas import tpu as pltpu
from jax.experimental.pallas import tpu_sc as plsc

B = 4096
F = 26
D = 32
NC, NS = 2, 16
NW = NC * NS
ROWS_W = B // NW
NROW = 1000000


@functools.partial(
    pl.kernel,
    out_type=jax.ShapeDtypeStruct((B, D), jnp.float32),
    mesh=plsc.VectorSubcoreMesh(core_axis_name="c", subcore_axis_name="s"),
    scratch_types=[
        pltpu.VMEM((F, ROWS_W), jnp.int32),
        pltpu.VMEM((ROWS_W, D), jnp.bfloat16),
        pltpu.VMEM((ROWS_W, D), jnp.float32),
        pltpu.SemaphoreType.DMA,
    ],
    compiler_params=pltpu.CompilerParams(use_tc_tiling_on_sc=False),
)
def _field_embed(xt_hbm, table_hbm, out_hbm, idx_v, acc_v, res_v, sem):
    wid = lax.axis_index("s") * NC + lax.axis_index("c")
    base = wid * ROWS_W
    pltpu.sync_copy(xt_hbm.at[:, pl.ds(base, ROWS_W)], idx_v)
    zeros = jnp.zeros((32,), jnp.bfloat16)

    def zrow(r, carry):
        acc_v[r] = zeros
        return carry

    lax.fori_loop(0, ROWS_W, zrow, 0)
    copies = []
    for f in range(F):
        copies.append(
            pltpu.async_copy(table_hbm.at[idx_v.at[f]], acc_v, sem, add=True)
        )
    for cp in copies:
        cp.wait()

    def crow(r, carry):
        res_v[r, pl.ds(0, 16)] = acc_v[r, pl.ds(0, 16)].astype(jnp.float32)
        res_v[r, pl.ds(16, 16)] = acc_v[r, pl.ds(16, 16)].astype(jnp.float32)
        return carry

    lax.fori_loop(0, ROWS_W, crow, 0)
    pltpu.sync_copy(res_v, out_hbm.at[pl.ds(base, ROWS_W)])


TWB = 8192
STEP = 4 * TWB
TGRID = -(-NROW // STEP)
GROWS = TGRID * TWB


def _transpose_body(in_ref, out_ref):
    out_ref[...] = jnp.concatenate(
        [in_ref[:, b * TWB:(b + 1) * TWB] for b in range(4)], axis=0
    ).astype(jnp.bfloat16).T


_table_rowmajor = pl.pallas_call(
    _transpose_body,
    out_shape=jax.ShapeDtypeStruct((GROWS, 4 * D), jnp.bfloat16),
    grid=(TGRID,),
    in_specs=[pl.BlockSpec((D, STEP), lambda i: (0, i))],
    out_specs=pl.BlockSpec((TWB, 4 * D), lambda i: (i, 0)),
)


def kernel(x, table):
    xt = x.T.astype(jnp.int32)
    rho = ((xt >> 15) << 15) + ((xt & 8191) << 2) + ((xt >> 13) & 3)
    t2 = _table_rowmajor(table.T)
    tr = jnp.reshape(t2, (4 * GROWS, D))
    return _field_embed(rho, tr)


# speedup vs baseline: 2.7319x; 2.7319x over previous
import functools

import jax
import jax.numpy as jnp
from jax import lax
from jax.experimental import pallas as pl
from jax.experimental.pallas import tpu as pltpu
from jax.experimental.pallas import tpu_sc as plsc

B = 4096
F = 26
D = 32
NC, NS = 2, 16
NW = NC * NS
ROWS_W = B // NW
NROW = 1000000


@functools.partial(
    pl.kernel,
    out_type=jax.ShapeDtypeStruct((B, D), jnp.float32),
    mesh=plsc.VectorSubcoreMesh(core_axis_name="c", subcore_axis_name="s"),
    scratch_types=[
        pltpu.VMEM((F, ROWS_W), jnp.int32),
        pltpu.VMEM((ROWS_W, D), jnp.float32),
        pltpu.SemaphoreType.DMA,
    ],
    compiler_params=pltpu.CompilerParams(use_tc_tiling_on_sc=False),
)
def _field_embed(xt_hbm, table_hbm, out_hbm, idx_v, acc_v, sem):
    wid = lax.axis_index("s") * NC + lax.axis_index("c")
    base = wid * ROWS_W
    pltpu.sync_copy(xt_hbm.at[:, pl.ds(base, ROWS_W)], idx_v)
    zeros = jnp.zeros((16,), jnp.float32)

    def zrow(r, carry):
        acc_v[r, pl.ds(0, 16)] = zeros
        acc_v[r, pl.ds(16, 16)] = zeros
        return carry

    lax.fori_loop(0, ROWS_W, zrow, 0)
    copies = []
    for f in range(F):
        copies.append(
            pltpu.async_copy(table_hbm.at[idx_v.at[f]], acc_v, sem, add=True)
        )
    for cp in copies:
        cp.wait()
    pltpu.sync_copy(acc_v, out_hbm.at[pl.ds(base, ROWS_W)])


TWB = 8192
STEP = 4 * TWB
TGRID = -(-NROW // STEP)
GROWS = TGRID * TWB


def _transpose_body(in_ref, out_ref):
    out_ref[...] = jnp.concatenate(
        [in_ref[:, b * TWB:(b + 1) * TWB] for b in range(4)], axis=0
    ).T


_table_rowmajor = pl.pallas_call(
    _transpose_body,
    out_shape=jax.ShapeDtypeStruct((GROWS, 4 * D), jnp.float32),
    grid=(TGRID,),
    in_specs=[pl.BlockSpec((D, STEP), lambda i: (0, i))],
    out_specs=pl.BlockSpec((TWB, 4 * D), lambda i: (i, 0)),
)


def kernel(x, table):
    xt = x.T.astype(jnp.int32)
    rho = ((xt >> 15) << 15) + ((xt & 8191) << 2) + ((xt >> 13) & 3)
    t2 = _table_rowmajor(table.T)
    tr = jnp.reshape(t2, (4 * GROWS, D))
    return _field_embed(rho, tr)


# speedup vs baseline: 2.7749x; 1.0157x over previous
import functools

import jax
import jax.numpy as jnp
from jax import lax
from jax.experimental import pallas as pl
from jax.experimental.pallas import tpu as pltpu
from jax.experimental.pallas import tpu_sc as plsc

B = 4096
F = 26
D = 32
NC, NS = 2, 16
NW = NC * NS
ROWS_W = B // NW
NROW = 1000000


@functools.partial(
    pl.kernel,
    out_type=jax.ShapeDtypeStruct((B, D), jnp.float32),
    mesh=plsc.VectorSubcoreMesh(core_axis_name="c", subcore_axis_name="s"),
    scratch_types=[
        pltpu.VMEM((F, ROWS_W), jnp.int32),
        pltpu.VMEM((ROWS_W, D), jnp.float32),
        pltpu.SemaphoreType.DMA,
    ],
    compiler_params=pltpu.CompilerParams(use_tc_tiling_on_sc=False),
)
def _field_embed(xt_hbm, table_hbm, out_hbm, idx_v, acc_v, sem):
    wid = lax.axis_index("s") * NC + lax.axis_index("c")
    base = wid * ROWS_W
    pltpu.sync_copy(xt_hbm.at[:, pl.ds(base, ROWS_W)], idx_v)
    zeros = jnp.zeros((16,), jnp.float32)

    def zrow(r, carry):
        acc_v[r, pl.ds(0, 16)] = zeros
        acc_v[r, pl.ds(16, 16)] = zeros
        return carry

    lax.fori_loop(0, ROWS_W, zrow, 0)
    copies = []
    for f in range(F):
        copies.append(
            pltpu.async_copy(table_hbm.at[idx_v.at[f]], acc_v, sem, add=True)
        )
    for cp in copies:
        cp.wait()
    pltpu.sync_copy(acc_v, out_hbm.at[pl.ds(base, ROWS_W)])


TWB = 16384
STEP = 4 * TWB
TGRID = -(-NROW // STEP)
GROWS = TGRID * TWB


def _transpose_body(in_ref, out_ref):
    out_ref[...] = jnp.concatenate(
        [in_ref[:, b * TWB:(b + 1) * TWB] for b in range(4)], axis=0
    ).T


_table_rowmajor = pl.pallas_call(
    _transpose_body,
    out_shape=jax.ShapeDtypeStruct((GROWS, 4 * D), jnp.float32),
    grid=(TGRID,),
    in_specs=[pl.BlockSpec((D, STEP), lambda i: (0, i))],
    out_specs=pl.BlockSpec((TWB, 4 * D), lambda i: (i, 0)),
)


def kernel(x, table):
    xt = x.T.astype(jnp.int32)
    rho = ((xt >> 16) << 16) + ((xt & 16383) << 2) + ((xt >> 14) & 3)
    t2 = _table_rowmajor(table.T)
    tr = jnp.reshape(t2, (4 * GROWS, D))
    return _field_embed(rho, tr)
